# Initial kernel scaffold; baseline (speedup 1.0000x reference)
import jax
import jax.numpy as jnp
from jax.experimental import pallas as pl


def kernel(x, edge_index, W1, b1, W2, b2):
    raise NotImplementedError("write your pallas kernel here")


# speedup vs baseline: 7.4351x; 7.4351x over previous
import jax
import jax.numpy as jnp
from jax import lax
from jax.experimental import pallas as pl
from jax.experimental.pallas import tpu as pltpu
from jax.experimental.pallas import tpu_sc as plsc

NN = 10000
EE = 320000
DD = 128
NC = 2
NS = 16
NW = NC * NS
K = 80
EPT = EE // NS
NCHUNK = EPT // K
EPW = EE // NW
NP = 10240
HNP = 5120
DUMP = 80
ACCR = HNP + DUMP
AIOC = ACCR // K
RPT = HNP // NS
RBC = RPT // K

_SC_CACHE = {}


def _mesh():
    return plsc.VectorSubcoreMesh(
        core_axis_name="c", subcore_axis_name="s",
        num_cores=NC, num_subcores=NS)


def _deg_body(dst_hbm, zeros_hbm, out_hbm, di_v, hist_v):
    c = lax.axis_index("c")
    s = lax.axis_index("s")
    w = s * NC + c
    base = w * EPW
    pltpu.sync_copy(zeros_hbm, hist_v)

    def body(i, carry):
        off = pl.multiple_of(base + i * K, 8)
        pltpu.sync_copy(dst_hbm.at[pl.ds(off, K)], di_v)
        for j in range(K // 16):
            d = di_v[pl.ds(j * 16, 16)]
            cnt, last = plsc.scan_count(d)
            plsc.addupdate_scatter(
                hist_v, [d], cnt.astype(jnp.float32), mask=last)
        return carry

    lax.fori_loop(0, EPW // K, body, 0)
    pltpu.sync_copy(hist_v, out_hbm.at[pl.ds(w * NP, NP)])


def _agg_body(src_hbm, dst_hbm, g_hbm, iota_hbm, zeros_hbm, out_hbm,
              si_v, di_v, dm_v, io_v, rows_v, zb_v, acc_sh, sem):
    c = lax.axis_index("c")
    s = lax.axis_index("s")
    pltpu.sync_copy(zeros_hbm, zb_v)
    for z in range(5):
        j = s + NS * z

        @pl.when(j < AIOC)
        def _():
            pltpu.sync_copy(iota_hbm.at[pl.ds(j * K, K)], io_v)
            pltpu.sync_copy(zb_v, acc_sh.at[io_v])

    plsc.subcore_barrier()
    lo = c * HNP

    def body(i, carry):
        off = pl.multiple_of(s * EPT + i * K, 8)
        pltpu.sync_copy(src_hbm.at[pl.ds(off, K)], si_v)
        pltpu.sync_copy(dst_hbm.at[pl.ds(off, K)], di_v)
        for j in range(K // 16):
            d = di_v[pl.ds(j * 16, 16)]
            local = d - lo
            ok = (local >= 0) & (local < HNP)
            dump = HNP + j * 16 + lax.iota(jnp.int32, 16)
            dm_v[pl.ds(j * 16, 16)] = jnp.where(ok, local, dump)
        pltpu.async_copy(g_hbm.at[si_v], rows_v, sem).wait()
        pltpu.sync_copy(rows_v, acc_sh.at[dm_v], add=True)
        return carry

    lax.fori_loop(0, NCHUNK, body, 0)
    plsc.subcore_barrier()
    for z in range(RBC):
        off = s * RPT + z * K
        pltpu.sync_copy(iota_hbm.at[pl.ds(off, K)], io_v)
        pltpu.async_copy(acc_sh.at[io_v], rows_v, sem).wait()
        pltpu.sync_copy(rows_v, out_hbm.at[c, pl.ds(off, K)])


def _sc_kernels():
    if "deg" not in _SC_CACHE:
        _SC_CACHE["deg"] = pl.kernel(
            _deg_body,
            out_type=jax.ShapeDtypeStruct((NW * NP,), jnp.float32),
            mesh=_mesh(),
            compiler_params=pltpu.CompilerParams(needs_layout_passes=False),
            scratch_types=[
                pltpu.VMEM((K,), jnp.int32),
                pltpu.VMEM((NP,), jnp.float32),
            ])
        _SC_CACHE["agg"] = pl.kernel(
            _agg_body,
            out_type=jax.ShapeDtypeStruct((NC, HNP, DD), jnp.float32),
            mesh=_mesh(),
            scratch_types=[
                pltpu.VMEM((K,), jnp.int32),
                pltpu.VMEM((K,), jnp.int32),
                pltpu.VMEM((K,), jnp.int32),
                pltpu.VMEM((K,), jnp.int32),
                pltpu.VMEM((K, DD), jnp.float32),
                pltpu.VMEM((K, DD), jnp.float32),
                pltpu.VMEM_SHARED((ACCR, DD), jnp.float32),
                pltpu.SemaphoreType.DMA,
            ])
    return _SC_CACHE["deg"], _SC_CACHE["agg"]


BLK = 1024
NB = NP // BLK


def _dinv_of(deg_ref):
    degp = deg_ref[...].reshape(NW, BLK)
    ones = jnp.ones((NW, 1), jnp.float32)
    deg = lax.dot_general(degp, ones, (((0,), (0,)), ((), ())),
                          preferred_element_type=jnp.float32)
    return lax.rsqrt(deg + 1.0)


def _lin1_body(deg_ref, x_ref, w_ref, g_ref):
    dinv = _dinv_of(deg_ref)
    h = jnp.dot(x_ref[...], w_ref[...], preferred_element_type=jnp.float32)
    g_ref[...] = h * dinv


def _mid_body(deg_ref, p_ref, g1_ref, b1_ref, w_ref, g2_ref):
    dinv = _dinv_of(deg_ref)
    ssum = p_ref[0] + g1_ref[...]
    a = jnp.maximum(ssum * dinv + b1_ref[...], 0.0)
    g2_ref[...] = jnp.dot(a, w_ref[...],
                          preferred_element_type=jnp.float32) * dinv


def _fin_body(deg_ref, p_ref, g2_ref, b2_ref, o_ref):
    dinv = _dinv_of(deg_ref)
    ssum = p_ref[0] + g2_ref[...]
    o_ref[...] = jnp.tanh(ssum * dinv + b2_ref[...])


_deg_spec = pl.BlockSpec((NW, 1, BLK), lambda i: (0, 0, i))
_row_spec = pl.BlockSpec((BLK, DD), lambda i: (i, 0))
_p_spec = pl.BlockSpec((1, BLK, DD), lambda i: (i // 5, i % 5, 0))
_w_spec = pl.BlockSpec((DD, DD), lambda i: (0, 0))
_b_spec = pl.BlockSpec((1, DD), lambda i: (0, 0))

_row_shape = jax.ShapeDtypeStruct((NN, DD), jnp.float32)

_lin1 = pl.pallas_call(
    _lin1_body, grid=(NB,),
    in_specs=[_deg_spec, _row_spec, _w_spec],
    out_specs=_row_spec,
    out_shape=_row_shape)

_mid = pl.pallas_call(
    _mid_body, grid=(NB,),
    in_specs=[_deg_spec, _p_spec, _row_spec, _b_spec, _w_spec],
    out_specs=_row_spec,
    out_shape=_row_shape)

_fin = pl.pallas_call(
    _fin_body, grid=(NB,),
    in_specs=[_deg_spec, _p_spec, _row_spec, _b_spec],
    out_specs=_row_spec,
    out_shape=_row_shape)


def kernel(x, edge_index, W1, b1, W2, b2):
    src = edge_index[0]
    dst = edge_index[1]
    zeros_np = jnp.zeros((NP,), jnp.float32)
    zeros_kd = jnp.zeros((K, DD), jnp.float32)
    iota_acc = jnp.arange(ACCR, dtype=jnp.int32)
    b1r = b1.reshape(1, DD)
    b2r = b2.reshape(1, DD)
    _deg, _agg = _sc_kernels()

    degp = _deg(dst, zeros_np).reshape(NW, 1, NP)
    g1 = _lin1(degp, x, W1)
    p1 = _agg(src, dst, g1, iota_acc, zeros_kd)
    g2 = _mid(degp, p1, g1, b1r, W2)
    p2 = _agg(src, dst, g2, iota_acc, zeros_kd)
    return _fin(degp, p2, g2, b2r)


# speedup vs baseline: 14.9496x; 2.0107x over previous
import jax
import jax.numpy as jnp
from jax import lax
from jax.experimental import pallas as pl
from jax.experimental.pallas import tpu as pltpu
from jax.experimental.pallas import tpu_sc as plsc

NN = 10000
EE = 320000
DD = 128
NC = 2
NS = 16
NW = NC * NS
K = 80
EPT = EE // NS
NCHUNK = EPT // K
EPW = EE // NW
NP = 10240
HNP = 5120
DUMP = 80
ACCR = HNP + DUMP
AIOC = ACCR // K
RPT = HNP // NS
RBC = RPT // K

_SC_CACHE = {}


def _mesh():
    return plsc.VectorSubcoreMesh(
        core_axis_name="c", subcore_axis_name="s",
        num_cores=NC, num_subcores=NS)


def _deg_body(dst_hbm, zeros_hbm, out_hbm, di_v, hist_v):
    c = lax.axis_index("c")
    s = lax.axis_index("s")
    w = s * NC + c
    base = w * EPW
    pltpu.sync_copy(zeros_hbm, hist_v)

    def body(i, carry):
        off = pl.multiple_of(base + i * K, 8)
        pltpu.sync_copy(dst_hbm.at[pl.ds(off, K)], di_v)
        for j in range(K // 16):
            d = di_v[pl.ds(j * 16, 16)]
            cnt, last = plsc.scan_count(d)
            plsc.addupdate_scatter(
                hist_v, [d], cnt.astype(jnp.float32), mask=last)
        return carry

    lax.fori_loop(0, EPW // K, body, 0)
    pltpu.sync_copy(hist_v, out_hbm.at[pl.ds(w * NP, NP)])


SUPER = 10
EDGES_SUPER = SUPER * K
NSUPER = EPT // EDGES_SUPER


def _agg_body(src_hbm, dst_hbm, g_hbm, iota_hbm, zeros_hbm, out_hbm,
              si_big, di_big, si_r0, si_r1, dm_r0, dm_r1, io_v,
              rows_r0, rows_r1, zb_v, acc_sh,
              sem_g0, sem_g1, sem_s0, sem_s1):
    c = lax.axis_index("c")
    s = lax.axis_index("s")
    si_r = (si_r0, si_r1)
    dm_r = (dm_r0, dm_r1)
    rows_r = (rows_r0, rows_r1)
    sem_g = (sem_g0, sem_g1)
    sem_s = (sem_s0, sem_s1)
    pltpu.sync_copy(zeros_hbm, zb_v)
    for z in range(5):
        j = s + NS * z

        @pl.when(j < AIOC)
        def _():
            pltpu.sync_copy(iota_hbm.at[pl.ds(j * K, K)], io_v)
            pltpu.sync_copy(zb_v, acc_sh.at[io_v])

    plsc.subcore_barrier()
    lo = c * HNP

    def prep(j, b):
        for q in range(K // 16):
            base_q = j * K + q * 16
            si_r[b][pl.ds(q * 16, 16)] = si_big[pl.ds(base_q, 16)]
            d = di_big[pl.ds(base_q, 16)]
            local = d - lo
            ok = (local >= 0) & (local < HNP)
            dump = HNP + ((j + q) % 5) * 16 + lax.iota(jnp.int32, 16)
            dm_r[b][pl.ds(q * 16, 16)] = jnp.where(ok, local, dump)

    def body(u, carry):
        off = pl.multiple_of(s * EPT + u * EDGES_SUPER, 8)
        pltpu.sync_copy(src_hbm.at[pl.ds(off, EDGES_SUPER)], si_big)
        pltpu.sync_copy(dst_hbm.at[pl.ds(off, EDGES_SUPER)], di_big)
        gd = [None, None]
        sd = [None, None]
        for j in range(SUPER):
            b = j % 2
            if j >= 2:
                sd[b].wait()
            prep(j, b)
            gd[b] = pltpu.async_copy(g_hbm.at[si_r[b]], rows_r[b], sem_g[b])
            if j >= 1:
                pb = (j - 1) % 2
                gd[pb].wait()
                sd[pb] = pltpu.async_copy(
                    rows_r[pb], acc_sh.at[dm_r[pb]], sem_s[pb], add=True)
        lb = (SUPER - 1) % 2
        gd[lb].wait()
        sd[lb] = pltpu.async_copy(
            rows_r[lb], acc_sh.at[dm_r[lb]], sem_s[lb], add=True)
        sd[0].wait()
        sd[1].wait()
        return carry

    lax.fori_loop(0, NSUPER, body, 0)
    plsc.subcore_barrier()
    for z in range(RBC):
        off = s * RPT + z * K
        pltpu.sync_copy(iota_hbm.at[pl.ds(off, K)], io_v)
        pltpu.async_copy(acc_sh.at[io_v], rows_r0, sem_g0).wait()
        pltpu.sync_copy(rows_r0, out_hbm.at[c, pl.ds(off, K)])


def _sc_kernels():
    if "deg" not in _SC_CACHE:
        _SC_CACHE["deg"] = pl.kernel(
            _deg_body,
            out_type=jax.ShapeDtypeStruct((NW * NP,), jnp.float32),
            mesh=_mesh(),
            compiler_params=pltpu.CompilerParams(needs_layout_passes=False),
            scratch_types=[
                pltpu.VMEM((K,), jnp.int32),
                pltpu.VMEM((NP,), jnp.float32),
            ])
        _SC_CACHE["agg"] = pl.kernel(
            _agg_body,
            out_type=jax.ShapeDtypeStruct((NC, HNP, DD), jnp.float32),
            mesh=_mesh(),
            scratch_types=[
                pltpu.VMEM((EDGES_SUPER,), jnp.int32),
                pltpu.VMEM((EDGES_SUPER,), jnp.int32),
                pltpu.VMEM((K,), jnp.int32),
                pltpu.VMEM((K,), jnp.int32),
                pltpu.VMEM((K,), jnp.int32),
                pltpu.VMEM((K,), jnp.int32),
                pltpu.VMEM((K,), jnp.int32),
                pltpu.VMEM((K, DD), jnp.float32),
                pltpu.VMEM((K, DD), jnp.float32),
                pltpu.VMEM((K, DD), jnp.float32),
                pltpu.VMEM_SHARED((ACCR, DD), jnp.float32),
                pltpu.SemaphoreType.DMA,
                pltpu.SemaphoreType.DMA,
                pltpu.SemaphoreType.DMA,
                pltpu.SemaphoreType.DMA,
            ])
    return _SC_CACHE["deg"], _SC_CACHE["agg"]


BLK = 1024
NB = NP // BLK


def _dinv_of(deg_ref):
    degp = deg_ref[...].reshape(NW, BLK)
    ones = jnp.ones((NW, 1), jnp.float32)
    deg = lax.dot_general(degp, ones, (((0,), (0,)), ((), ())),
                          preferred_element_type=jnp.float32)
    return lax.rsqrt(deg + 1.0)


def _lin1_body(deg_ref, x_ref, w_ref, g_ref):
    dinv = _dinv_of(deg_ref)
    h = jnp.dot(x_ref[...], w_ref[...], preferred_element_type=jnp.float32)
    g_ref[...] = h * dinv


def _mid_body(deg_ref, p_ref, g1_ref, b1_ref, w_ref, g2_ref):
    dinv = _dinv_of(deg_ref)
    ssum = p_ref[0] + g1_ref[...]
    a = jnp.maximum(ssum * dinv + b1_ref[...], 0.0)
    g2_ref[...] = jnp.dot(a, w_ref[...],
                          preferred_element_type=jnp.float32) * dinv


def _fin_body(deg_ref, p_ref, g2_ref, b2_ref, o_ref):
    dinv = _dinv_of(deg_ref)
    ssum = p_ref[0] + g2_ref[...]
    o_ref[...] = jnp.tanh(ssum * dinv + b2_ref[...])


_deg_spec = pl.BlockSpec((NW, 1, BLK), lambda i: (0, 0, i))
_row_spec = pl.BlockSpec((BLK, DD), lambda i: (i, 0))
_p_spec = pl.BlockSpec((1, BLK, DD), lambda i: (i // 5, i % 5, 0))
_w_spec = pl.BlockSpec((DD, DD), lambda i: (0, 0))
_b_spec = pl.BlockSpec((1, DD), lambda i: (0, 0))

_row_shape = jax.ShapeDtypeStruct((NN, DD), jnp.float32)

_lin1 = pl.pallas_call(
    _lin1_body, grid=(NB,),
    in_specs=[_deg_spec, _row_spec, _w_spec],
    out_specs=_row_spec,
    out_shape=_row_shape)

_mid = pl.pallas_call(
    _mid_body, grid=(NB,),
    in_specs=[_deg_spec, _p_spec, _row_spec, _b_spec, _w_spec],
    out_specs=_row_spec,
    out_shape=_row_shape)

_fin = pl.pallas_call(
    _fin_body, grid=(NB,),
    in_specs=[_deg_spec, _p_spec, _row_spec, _b_spec],
    out_specs=_row_spec,
    out_shape=_row_shape)


def kernel(x, edge_index, W1, b1, W2, b2):
    src = edge_index[0]
    dst = edge_index[1]
    zeros_np = jnp.zeros((NP,), jnp.float32)
    zeros_kd = jnp.zeros((K, DD), jnp.float32)
    iota_acc = jnp.arange(ACCR, dtype=jnp.int32)
    b1r = b1.reshape(1, DD)
    b2r = b2.reshape(1, DD)
    _deg, _agg = _sc_kernels()

    degp = _deg(dst, zeros_np).reshape(NW, 1, NP)
    g1 = _lin1(degp, x, W1)
    p1 = _agg(src, dst, g1, iota_acc, zeros_kd)
    g2 = _mid(degp, p1, g1, b1r, W2)
    p2 = _agg(src, dst, g2, iota_acc, zeros_kd)
    return _fin(degp, p2, g2, b2r)


# speedup vs baseline: 16.7315x; 1.1192x over previous
import jax
import jax.numpy as jnp
from jax import lax
from jax.experimental import pallas as pl
from jax.experimental.pallas import tpu as pltpu
from jax.experimental.pallas import tpu_sc as plsc

NN = 10000
EE = 320000
DD = 128
NC = 2
NS = 16
NW = NC * NS
K = 80
EPT = EE // NS
NCHUNK = EPT // K
EPW = EE // NW
NP = 10240
HNP = 5120
DUMP = 80
ACCR = HNP + DUMP
AIOC = ACCR // K
RPT = HNP // NS
RBC = RPT // K

_SC_CACHE = {}


def _mesh():
    return plsc.VectorSubcoreMesh(
        core_axis_name="c", subcore_axis_name="s",
        num_cores=NC, num_subcores=NS)


def _deg_body(dst_hbm, zeros_hbm, out_hbm, di_v, hist_v):
    c = lax.axis_index("c")
    s = lax.axis_index("s")
    w = s * NC + c
    base = w * EPW
    pltpu.sync_copy(zeros_hbm, hist_v)

    def body(i, carry):
        off = pl.multiple_of(base + i * K, 8)
        pltpu.sync_copy(dst_hbm.at[pl.ds(off, K)], di_v)
        for j in range(K // 16):
            d = di_v[pl.ds(j * 16, 16)]
            cnt, last = plsc.scan_count(d)
            plsc.addupdate_scatter(
                hist_v, [d], cnt.astype(jnp.float32), mask=last)
        return carry

    lax.fori_loop(0, EPW // K, body, 0)
    pltpu.sync_copy(hist_v, out_hbm.at[pl.ds(w * NP, NP)])


SUPER = 25
EDGES_SUPER = SUPER * K
NSUPER = EPT // EDGES_SUPER
NSLOT = 4


def _agg_body(src_hbm, dst_hbm, g_hbm, iota_hbm, zeros_hbm, out_hbm,
              si_big, di_big,
              si_r0, si_r1, si_r2, si_r3,
              dm_r0, dm_r1, dm_r2, dm_r3, io_v,
              rows_r0, rows_r1, rows_r2, rows_r3, zb_v, acc_sh,
              sem_g0, sem_g1, sem_g2, sem_g3,
              sem_s0, sem_s1, sem_s2, sem_s3):
    c = lax.axis_index("c")
    s = lax.axis_index("s")
    si_r = (si_r0, si_r1, si_r2, si_r3)
    dm_r = (dm_r0, dm_r1, dm_r2, dm_r3)
    rows_r = (rows_r0, rows_r1, rows_r2, rows_r3)
    sem_g = (sem_g0, sem_g1, sem_g2, sem_g3)
    sem_s = (sem_s0, sem_s1, sem_s2, sem_s3)
    pltpu.sync_copy(zeros_hbm, zb_v)
    for z in range(5):
        j = s + NS * z

        @pl.when(j < AIOC)
        def _():
            pltpu.sync_copy(iota_hbm.at[pl.ds(j * K, K)], io_v)
            pltpu.sync_copy(zb_v, acc_sh.at[io_v])

    plsc.subcore_barrier()
    lo = c * HNP

    def prep(j, b):
        for q in range(K // 16):
            base_q = j * K + q * 16
            si_r[b][pl.ds(q * 16, 16)] = si_big[pl.ds(base_q, 16)]
            d = di_big[pl.ds(base_q, 16)]
            local = d - lo
            ok = (local >= 0) & (local < HNP)
            dump = HNP + ((j + q) % 5) * 16 + lax.iota(jnp.int32, 16)
            dm_r[b][pl.ds(q * 16, 16)] = jnp.where(ok, local, dump)

    def body(u, carry):
        off = pl.multiple_of(s * EPT + u * EDGES_SUPER, 8)
        pltpu.sync_copy(src_hbm.at[pl.ds(off, EDGES_SUPER)], si_big)
        pltpu.sync_copy(dst_hbm.at[pl.ds(off, EDGES_SUPER)], di_big)
        gd = [None] * NSLOT
        sd = [None] * NSLOT
        for j in range(SUPER):
            b = j % NSLOT
            if j >= NSLOT:
                sd[b].wait()
            prep(j, b)
            gd[b] = pltpu.async_copy(g_hbm.at[si_r[b]], rows_r[b], sem_g[b])
            if j >= 1:
                pb = (j - 1) % NSLOT
                gd[pb].wait()
                sd[pb] = pltpu.async_copy(
                    rows_r[pb], acc_sh.at[dm_r[pb]], sem_s[pb], add=True)
        lb = (SUPER - 1) % NSLOT
        gd[lb].wait()
        sd[lb] = pltpu.async_copy(
            rows_r[lb], acc_sh.at[dm_r[lb]], sem_s[lb], add=True)
        for b in range(NSLOT):
            if sd[b] is not None:
                sd[b].wait()
        return carry

    lax.fori_loop(0, NSUPER, body, 0)
    plsc.subcore_barrier()
    for z in range(RBC):
        off = s * RPT + z * K
        pltpu.sync_copy(iota_hbm.at[pl.ds(off, K)], io_v)
        pltpu.async_copy(acc_sh.at[io_v], rows_r0, sem_g0).wait()
        pltpu.sync_copy(rows_r0, out_hbm.at[c, pl.ds(off, K)])


def _sc_kernels():
    if "deg" not in _SC_CACHE:
        _SC_CACHE["deg"] = pl.kernel(
            _deg_body,
            out_type=jax.ShapeDtypeStruct((NW * NP,), jnp.float32),
            mesh=_mesh(),
            compiler_params=pltpu.CompilerParams(needs_layout_passes=False),
            scratch_types=[
                pltpu.VMEM((K,), jnp.int32),
                pltpu.VMEM((NP,), jnp.float32),
            ])
        _SC_CACHE["agg"] = pl.kernel(
            _agg_body,
            out_type=jax.ShapeDtypeStruct((NC, HNP, DD), jnp.float32),
            mesh=_mesh(),
            scratch_types=[
                pltpu.VMEM((EDGES_SUPER,), jnp.int32),
                pltpu.VMEM((EDGES_SUPER,), jnp.int32),
                *[pltpu.VMEM((K,), jnp.int32) for _ in range(NSLOT)],
                *[pltpu.VMEM((K,), jnp.int32) for _ in range(NSLOT)],
                pltpu.VMEM((K,), jnp.int32),
                *[pltpu.VMEM((K, DD), jnp.float32) for _ in range(NSLOT)],
                pltpu.VMEM((K, DD), jnp.float32),
                pltpu.VMEM_SHARED((ACCR, DD), jnp.float32),
                *[pltpu.SemaphoreType.DMA for _ in range(2 * NSLOT)],
            ])
    return _SC_CACHE["deg"], _SC_CACHE["agg"]


BLK = 1024
NB = NP // BLK


def _dinv_of(deg_ref):
    degp = deg_ref[...].reshape(NW, BLK)
    ones = jnp.ones((NW, 1), jnp.float32)
    deg = lax.dot_general(degp, ones, (((0,), (0,)), ((), ())),
                          preferred_element_type=jnp.float32)
    return lax.rsqrt(deg + 1.0)


def _lin1_body(deg_ref, x_ref, w_ref, g_ref):
    dinv = _dinv_of(deg_ref)
    h = jnp.dot(x_ref[...], w_ref[...], preferred_element_type=jnp.float32)
    g_ref[...] = h * dinv


def _mid_body(deg_ref, p_ref, g1_ref, b1_ref, w_ref, g2_ref):
    dinv = _dinv_of(deg_ref)
    ssum = p_ref[0] + g1_ref[...]
    a = jnp.maximum(ssum * dinv + b1_ref[...], 0.0)
    g2_ref[...] = jnp.dot(a, w_ref[...],
                          preferred_element_type=jnp.float32) * dinv


def _fin_body(deg_ref, p_ref, g2_ref, b2_ref, o_ref):
    dinv = _dinv_of(deg_ref)
    ssum = p_ref[0] + g2_ref[...]
    o_ref[...] = jnp.tanh(ssum * dinv + b2_ref[...])


_deg_spec = pl.BlockSpec((NW, 1, BLK), lambda i: (0, 0, i))
_row_spec = pl.BlockSpec((BLK, DD), lambda i: (i, 0))
_p_spec = pl.BlockSpec((1, BLK, DD), lambda i: (i // 5, i % 5, 0))
_w_spec = pl.BlockSpec((DD, DD), lambda i: (0, 0))
_b_spec = pl.BlockSpec((1, DD), lambda i: (0, 0))

_row_shape = jax.ShapeDtypeStruct((NN, DD), jnp.float32)

_lin1 = pl.pallas_call(
    _lin1_body, grid=(NB,),
    in_specs=[_deg_spec, _row_spec, _w_spec],
    out_specs=_row_spec,
    out_shape=_row_shape)

_mid = pl.pallas_call(
    _mid_body, grid=(NB,),
    in_specs=[_deg_spec, _p_spec, _row_spec, _b_spec, _w_spec],
    out_specs=_row_spec,
    out_shape=_row_shape)

_fin = pl.pallas_call(
    _fin_body, grid=(NB,),
    in_specs=[_deg_spec, _p_spec, _row_spec, _b_spec],
    out_specs=_row_spec,
    out_shape=_row_shape)


def kernel(x, edge_index, W1, b1, W2, b2):
    src = edge_index[0]
    dst = edge_index[1]
    zeros_np = jnp.zeros((NP,), jnp.float32)
    zeros_kd = jnp.zeros((K, DD), jnp.float32)
    iota_acc = jnp.arange(ACCR, dtype=jnp.int32)
    b1r = b1.reshape(1, DD)
    b2r = b2.reshape(1, DD)
    _deg, _agg = _sc_kernels()

    degp = _deg(dst, zeros_np).reshape(NW, 1, NP)
    g1 = _lin1(degp, x, W1)
    p1 = _agg(src, dst, g1, iota_acc, zeros_kd)
    g2 = _mid(degp, p1, g1, b1r, W2)
    p2 = _agg(src, dst, g2, iota_acc, zeros_kd)
    return _fin(degp, p2, g2, b2r)
